# Initial kernel scaffold; baseline (speedup 1.0000x reference)
import functools

import jax
import jax.numpy as jnp
from jax import lax
from jax.experimental import pallas as pl
from jax.experimental.pallas import tpu as pltpu
from jax.experimental.pallas import tpu_sc as plsc

N = 10000
E = 320000
D = 128
H = 256
O = 128

NS = 16
GRP = 128
NGROUPS = 157
E_PAD = NS * NGROUPS * GRP
ACC_ROWS = N + 8

ROWS_PER_TILE = N // NS


def _sc_agg(F):
    mesh = plsc.VectorSubcoreMesh(core_axis_name="c", subcore_axis_name="s")

    @functools.partial(
        pl.kernel,
        out_type=(
            jax.ShapeDtypeStruct((N, F), jnp.float32),
            jax.ShapeDtypeStruct((N, F), jnp.float32),
        ),
        mesh=mesh,
        scratch_types=[
            pltpu.VMEM((NGROUPS, GRP), jnp.int32),
            pltpu.VMEM((NGROUPS, GRP), jnp.int32),
            pltpu.VMEM((GRP, F), jnp.float32),
            pltpu.VMEM_SHARED((ACC_ROWS, F), jnp.float32),
            pltpu.SemaphoreType.DMA,
        ],
    )
    def agg(x0_hbm, x1_hbm, src_hbm, dst_hbm, out0_hbm, out1_hbm,
            src_v, dst_v, rows_v, acc, sem):
        c = lax.axis_index("c")
        s = lax.axis_index("s")
        rbase = s * ROWS_PER_TILE

        pltpu.sync_copy(src_hbm.at[s], src_v)
        pltpu.sync_copy(dst_hbm.at[s], dst_v)

        @pl.when(c == 0)
        def _():
            pltpu.sync_copy(x0_hbm.at[pl.ds(rbase, ROWS_PER_TILE)],
                            acc.at[pl.ds(rbase, ROWS_PER_TILE)])

        @pl.when(c == 1)
        def _():
            pltpu.sync_copy(x1_hbm.at[pl.ds(rbase, ROWS_PER_TILE)],
                            acc.at[pl.ds(rbase, ROWS_PER_TILE)])

        plsc.subcore_barrier()

        @pl.loop(0, NGROUPS)
        def _(j):
            @pl.when(c == 0)
            def _():
                pltpu.async_copy(x0_hbm.at[src_v.at[j]], rows_v, sem).wait()

            @pl.when(c == 1)
            def _():
                pltpu.async_copy(x1_hbm.at[src_v.at[j]], rows_v, sem).wait()

            pltpu.sync_copy(rows_v, acc.at[dst_v.at[j]], add=True)

        plsc.subcore_barrier()

        @pl.when(c == 0)
        def _():
            pltpu.sync_copy(acc.at[pl.ds(rbase, ROWS_PER_TILE)],
                            out0_hbm.at[pl.ds(rbase, ROWS_PER_TILE)])

        @pl.when(c == 1)
        def _():
            pltpu.sync_copy(acc.at[pl.ds(rbase, ROWS_PER_TILE)],
                            out1_hbm.at[pl.ds(rbase, ROWS_PER_TILE)])

    return agg


_sc_agg_64 = _sc_agg(64)
_sc_agg_128 = _sc_agg(128)

BN = 400


def _mlp1_body(a0_ref, a1_ref, w1_ref, b1_ref, w2_ref, b2_ref,
               h0_ref, h1_ref):
    agg = jnp.concatenate([a0_ref[...], a1_ref[...]], axis=1)
    z = jnp.dot(agg, w1_ref[...], preferred_element_type=jnp.float32)
    z = jnp.maximum(z + b1_ref[...], 0.0)
    h = jnp.dot(z, w2_ref[...], preferred_element_type=jnp.float32)
    h = h + b2_ref[...]
    h0_ref[...] = h[:, :H // 2]
    h1_ref[...] = h[:, H // 2:]


def _tc_mlp1(a0, a1, w1, b1, w2, b2):
    grid = (N // BN,)
    return pl.pallas_call(
        _mlp1_body,
        grid=grid,
        in_specs=[
            pl.BlockSpec((BN, D // 2), lambda i: (i, 0)),
            pl.BlockSpec((BN, D // 2), lambda i: (i, 0)),
            pl.BlockSpec((D, H), lambda i: (0, 0)),
            pl.BlockSpec((1, H), lambda i: (0, 0)),
            pl.BlockSpec((H, H), lambda i: (0, 0)),
            pl.BlockSpec((1, H), lambda i: (0, 0)),
        ],
        out_specs=[
            pl.BlockSpec((BN, H // 2), lambda i: (i, 0)),
            pl.BlockSpec((BN, H // 2), lambda i: (i, 0)),
        ],
        out_shape=[
            jax.ShapeDtypeStruct((N, H // 2), jnp.float32),
            jax.ShapeDtypeStruct((N, H // 2), jnp.float32),
        ],
    )(a0, a1, w1, b1, w2, b2)


def _mlp2_body(a0_ref, a1_ref, w1_ref, b1_ref, w2_ref, b2_ref,
               wr1_ref, br1_ref, wr2_ref, br2_ref, out_ref):
    agg = jnp.concatenate([a0_ref[...], a1_ref[...]], axis=1)
    z = jnp.dot(agg, w1_ref[...], preferred_element_type=jnp.float32)
    z = jnp.maximum(z + b1_ref[...], 0.0)
    h = jnp.dot(z, w2_ref[...], preferred_element_type=jnp.float32)
    h = h + b2_ref[...]
    z2 = jnp.dot(h, wr1_ref[...], preferred_element_type=jnp.float32)
    z2 = jnp.maximum(z2 + br1_ref[...], 0.0)
    out = jnp.dot(z2, wr2_ref[...], preferred_element_type=jnp.float32)
    out_ref[...] = out + br2_ref[...]


def _tc_mlp2(a0, a1, w1, b1, w2, b2, wr1, br1, wr2, br2):
    grid = (N // BN,)
    return pl.pallas_call(
        _mlp2_body,
        grid=grid,
        in_specs=[
            pl.BlockSpec((BN, H // 2), lambda i: (i, 0)),
            pl.BlockSpec((BN, H // 2), lambda i: (i, 0)),
            pl.BlockSpec((H, H), lambda i: (0, 0)),
            pl.BlockSpec((1, H), lambda i: (0, 0)),
            pl.BlockSpec((H, H), lambda i: (0, 0)),
            pl.BlockSpec((1, H), lambda i: (0, 0)),
            pl.BlockSpec((H, H), lambda i: (0, 0)),
            pl.BlockSpec((1, H), lambda i: (0, 0)),
            pl.BlockSpec((H, O), lambda i: (0, 0)),
            pl.BlockSpec((1, O), lambda i: (0, 0)),
        ],
        out_specs=pl.BlockSpec((BN, O), lambda i: (i, 0)),
        out_shape=jax.ShapeDtypeStruct((N, O), jnp.float32),
    )(a0, a1, w1, b1, w2, b2, wr1, br1, wr2, br2)


def kernel(x, edge_index, W1_0, b1_0, W2_0, b2_0, W1_1, b1_1, W2_1, b2_1,
           Wr1, br1, Wr2, br2):
    src = edge_index[0]
    dst = edge_index[1]
    pad = E_PAD - E
    src_p = jnp.concatenate(
        [src, jnp.zeros((pad,), jnp.int32)]).reshape(NS, NGROUPS, GRP)
    dst_p = jnp.concatenate(
        [dst, jnp.full((pad,), N, jnp.int32)]).reshape(NS, NGROUPS, GRP)

    x0 = x[:, :D // 2]
    x1 = x[:, D // 2:]

    a0_0, a0_1 = _sc_agg_64(x0, x1, src_p, dst_p)
    h0, h1 = _tc_mlp1(a0_0, a0_1, W1_0, b1_0.reshape(1, H),
                      W2_0, b2_0.reshape(1, H))
    a1_0, a1_1 = _sc_agg_128(h0, h1, src_p, dst_p)
    return _tc_mlp2(a1_0, a1_1, W1_1, b1_1.reshape(1, H),
                    W2_1, b2_1.reshape(1, H), Wr1, br1.reshape(1, H),
                    Wr2, br2.reshape(1, O))


# speedup vs baseline: 4.4340x; 4.4340x over previous
import functools

import jax
import jax.numpy as jnp
from jax import lax
from jax.experimental import pallas as pl
from jax.experimental.pallas import tpu as pltpu
from jax.experimental.pallas import tpu_sc as plsc

N = 10000
E = 320000
D = 128
H = 256
O = 128

NC = 2
NS = 16
GRP = 128
NGTOT = 2528
E_PAD = NGTOT * GRP
G_L0 = NGTOT // (NC * NS)
G_L1 = NGTOT // NS
G_PHASES = ((0, 80), (80, 78))
G_PHASE = 80
ACC_ROWS = N + 8

RCHUNK = 624
RTAIL = N - NS * RCHUNK
RTAIL_BASE = NS * RCHUNK

_MESH = plsc.VectorSubcoreMesh(core_axis_name="c", subcore_axis_name="s")


@functools.partial(
    pl.kernel,
    out_type=(
        jax.ShapeDtypeStruct((N, D), jnp.float32),
        jax.ShapeDtypeStruct((N, D), jnp.float32),
    ),
    mesh=_MESH,
    scratch_types=[
        pltpu.VMEM((G_L0, GRP), jnp.int32),
        pltpu.VMEM((G_L0, GRP), jnp.int32),
        pltpu.VMEM((GRP, D), jnp.float32),
        pltpu.VMEM_SHARED((ACC_ROWS, D), jnp.float32),
        pltpu.SemaphoreType.DMA,
    ],
)
def _sc_agg0(x_hbm, src_hbm, dst_hbm, out0_hbm, out1_hbm,
             src_v, dst_v, rows_v, acc, sem):
    c = lax.axis_index("c")
    s = lax.axis_index("s")
    w = c * NS + s
    rbase = s * RCHUNK

    pltpu.sync_copy(src_hbm.at[w], src_v)
    pltpu.sync_copy(dst_hbm.at[w], dst_v)

    pltpu.sync_copy(x_hbm.at[pl.ds(rbase, RCHUNK)],
                    acc.at[pl.ds(rbase, RCHUNK)])

    @pl.when(s == 0)
    def _():
        pltpu.sync_copy(x_hbm.at[pl.ds(RTAIL_BASE, RTAIL)],
                        acc.at[pl.ds(RTAIL_BASE, RTAIL)])

    plsc.subcore_barrier()

    @pl.loop(0, G_L0)
    def _(j):
        pltpu.async_copy(x_hbm.at[src_v.at[j]], rows_v, sem).wait()
        pltpu.sync_copy(rows_v, acc.at[dst_v.at[j]], add=True)

    plsc.subcore_barrier()

    @pl.when(c == 0)
    def _():
        pltpu.sync_copy(acc.at[pl.ds(rbase, RCHUNK)],
                        out0_hbm.at[pl.ds(rbase, RCHUNK)])

    @pl.when(c == 1)
    def _():
        pltpu.sync_copy(acc.at[pl.ds(rbase, RCHUNK)],
                        out1_hbm.at[pl.ds(rbase, RCHUNK)])

    @pl.when((c == 0) & (s == 0))
    def _():
        pltpu.sync_copy(acc.at[pl.ds(RTAIL_BASE, RTAIL)],
                        out0_hbm.at[pl.ds(RTAIL_BASE, RTAIL)])

    @pl.when((c == 1) & (s == 0))
    def _():
        pltpu.sync_copy(acc.at[pl.ds(RTAIL_BASE, RTAIL)],
                        out1_hbm.at[pl.ds(RTAIL_BASE, RTAIL)])


@functools.partial(
    pl.kernel,
    out_type=(
        jax.ShapeDtypeStruct((N, H // 2), jnp.float32),
        jax.ShapeDtypeStruct((N, H // 2), jnp.float32),
    ),
    mesh=_MESH,
    scratch_types=[
        pltpu.VMEM((G_PHASE, GRP), jnp.int32),
        pltpu.VMEM((G_PHASE, GRP), jnp.int32),
        pltpu.VMEM((GRP, H // 2), jnp.float32),
        pltpu.VMEM_SHARED((ACC_ROWS, H // 2), jnp.float32),
        pltpu.SemaphoreType.DMA,
    ],
)
def _sc_agg1(h0_hbm, h1_hbm, src_hbm, dst_hbm, out0_hbm, out1_hbm,
             src_v, dst_v, rows_v, acc, sem):
    c = lax.axis_index("c")
    s = lax.axis_index("s")
    rbase = s * RCHUNK

    @pl.when(c == 0)
    def _():
        pltpu.sync_copy(h0_hbm.at[pl.ds(rbase, RCHUNK)],
                        acc.at[pl.ds(rbase, RCHUNK)])

    @pl.when(c == 1)
    def _():
        pltpu.sync_copy(h1_hbm.at[pl.ds(rbase, RCHUNK)],
                        acc.at[pl.ds(rbase, RCHUNK)])

    @pl.when((c == 0) & (s == 0))
    def _():
        pltpu.sync_copy(h0_hbm.at[pl.ds(RTAIL_BASE, RTAIL)],
                        acc.at[pl.ds(RTAIL_BASE, RTAIL)])

    @pl.when((c == 1) & (s == 0))
    def _():
        pltpu.sync_copy(h1_hbm.at[pl.ds(RTAIL_BASE, RTAIL)],
                        acc.at[pl.ds(RTAIL_BASE, RTAIL)])

    plsc.subcore_barrier()

    for gbase, gcount in G_PHASES:
        pltpu.sync_copy(src_hbm.at[s, pl.ds(gbase, gcount)],
                        src_v.at[pl.ds(0, gcount)])
        pltpu.sync_copy(dst_hbm.at[s, pl.ds(gbase, gcount)],
                        dst_v.at[pl.ds(0, gcount)])

        @pl.loop(0, gcount)
        def _(j):
            @pl.when(c == 0)
            def _():
                pltpu.async_copy(h0_hbm.at[src_v.at[j]], rows_v, sem).wait()

            @pl.when(c == 1)
            def _():
                pltpu.async_copy(h1_hbm.at[src_v.at[j]], rows_v, sem).wait()

            pltpu.sync_copy(rows_v, acc.at[dst_v.at[j]], add=True)

    plsc.subcore_barrier()

    @pl.when(c == 0)
    def _():
        pltpu.sync_copy(acc.at[pl.ds(rbase, RCHUNK)],
                        out0_hbm.at[pl.ds(rbase, RCHUNK)])

    @pl.when(c == 1)
    def _():
        pltpu.sync_copy(acc.at[pl.ds(rbase, RCHUNK)],
                        out1_hbm.at[pl.ds(rbase, RCHUNK)])

    @pl.when((c == 0) & (s == 0))
    def _():
        pltpu.sync_copy(acc.at[pl.ds(RTAIL_BASE, RTAIL)],
                        out0_hbm.at[pl.ds(RTAIL_BASE, RTAIL)])

    @pl.when((c == 1) & (s == 0))
    def _():
        pltpu.sync_copy(acc.at[pl.ds(RTAIL_BASE, RTAIL)],
                        out1_hbm.at[pl.ds(RTAIL_BASE, RTAIL)])


BN = 400


def _mlp1_body(p0_ref, p1_ref, x_ref, w1_ref, b1_ref, w2_ref, b2_ref,
               h0_ref, h1_ref):
    agg = p0_ref[...] + p1_ref[...] - x_ref[...]
    z = jnp.dot(agg, w1_ref[...], preferred_element_type=jnp.float32)
    z = jnp.maximum(z + b1_ref[...], 0.0)
    h = jnp.dot(z, w2_ref[...], preferred_element_type=jnp.float32)
    h = h + b2_ref[...]
    h0_ref[...] = h[:, :H // 2]
    h1_ref[...] = h[:, H // 2:]


def _tc_mlp1(p0, p1, x, w1, b1, w2, b2):
    grid = (N // BN,)
    return pl.pallas_call(
        _mlp1_body,
        grid=grid,
        in_specs=[
            pl.BlockSpec((BN, D), lambda i: (i, 0)),
            pl.BlockSpec((BN, D), lambda i: (i, 0)),
            pl.BlockSpec((BN, D), lambda i: (i, 0)),
            pl.BlockSpec((D, H), lambda i: (0, 0)),
            pl.BlockSpec((1, H), lambda i: (0, 0)),
            pl.BlockSpec((H, H), lambda i: (0, 0)),
            pl.BlockSpec((1, H), lambda i: (0, 0)),
        ],
        out_specs=[
            pl.BlockSpec((BN, H // 2), lambda i: (i, 0)),
            pl.BlockSpec((BN, H // 2), lambda i: (i, 0)),
        ],
        out_shape=[
            jax.ShapeDtypeStruct((N, H // 2), jnp.float32),
            jax.ShapeDtypeStruct((N, H // 2), jnp.float32),
        ],
    )(p0, p1, x, w1, b1, w2, b2)


def _mlp2_body(a0_ref, a1_ref, w1_ref, b1_ref, w2_ref, b2_ref,
               wr1_ref, br1_ref, wr2_ref, br2_ref, out_ref):
    agg = jnp.concatenate([a0_ref[...], a1_ref[...]], axis=1)
    z = jnp.dot(agg, w1_ref[...], preferred_element_type=jnp.float32)
    z = jnp.maximum(z + b1_ref[...], 0.0)
    h = jnp.dot(z, w2_ref[...], preferred_element_type=jnp.float32)
    h = h + b2_ref[...]
    z2 = jnp.dot(h, wr1_ref[...], preferred_element_type=jnp.float32)
    z2 = jnp.maximum(z2 + br1_ref[...], 0.0)
    out = jnp.dot(z2, wr2_ref[...], preferred_element_type=jnp.float32)
    out_ref[...] = out + br2_ref[...]


def _tc_mlp2(a0, a1, w1, b1, w2, b2, wr1, br1, wr2, br2):
    grid = (N // BN,)
    return pl.pallas_call(
        _mlp2_body,
        grid=grid,
        in_specs=[
            pl.BlockSpec((BN, H // 2), lambda i: (i, 0)),
            pl.BlockSpec((BN, H // 2), lambda i: (i, 0)),
            pl.BlockSpec((H, H), lambda i: (0, 0)),
            pl.BlockSpec((1, H), lambda i: (0, 0)),
            pl.BlockSpec((H, H), lambda i: (0, 0)),
            pl.BlockSpec((1, H), lambda i: (0, 0)),
            pl.BlockSpec((H, H), lambda i: (0, 0)),
            pl.BlockSpec((1, H), lambda i: (0, 0)),
            pl.BlockSpec((H, O), lambda i: (0, 0)),
            pl.BlockSpec((1, O), lambda i: (0, 0)),
        ],
        out_specs=pl.BlockSpec((BN, O), lambda i: (i, 0)),
        out_shape=jax.ShapeDtypeStruct((N, O), jnp.float32),
    )(a0, a1, w1, b1, w2, b2, wr1, br1, wr2, br2)


def kernel(x, edge_index, W1_0, b1_0, W2_0, b2_0, W1_1, b1_1, W2_1, b2_1,
           Wr1, br1, Wr2, br2):
    src = edge_index[0]
    dst = edge_index[1]
    pad = E_PAD - E
    src_p = jnp.concatenate([src, jnp.zeros((pad,), jnp.int32)])
    dst_p = jnp.concatenate([dst, jnp.full((pad,), N, jnp.int32)])
    src_l0 = src_p.reshape(NC * NS, G_L0, GRP)
    dst_l0 = dst_p.reshape(NC * NS, G_L0, GRP)
    src_l1 = src_p.reshape(NS, G_L1, GRP)
    dst_l1 = dst_p.reshape(NS, G_L1, GRP)

    p0, p1 = _sc_agg0(x, src_l0, dst_l0)
    h0, h1 = _tc_mlp1(p0, p1, x, W1_0, b1_0.reshape(1, H),
                      W2_0, b2_0.reshape(1, H))
    a1_0, a1_1 = _sc_agg1(h0, h1, src_l1, dst_l1)
    return _tc_mlp2(a1_0, a1_1, W1_1, b1_1.reshape(1, H),
                    W2_1, b2_1.reshape(1, H), Wr1, br1.reshape(1, H),
                    Wr2, br2.reshape(1, O))
